# baseline (speedup 1.0000x reference)
import functools

import jax
import jax.numpy as jnp
from jax import lax
from jax.experimental import pallas as pl
from jax.experimental.pallas import tpu as pltpu
from jax.experimental.pallas import tpu_sc as plsc

_NC = 2
_NS = 16
_NW = _NC * _NS

_K = 4
_NBUF = 4


@functools.lru_cache(maxsize=None)
def _build_sc_gather(b_tot, v, h, k, nch):
    nbuf = _NBUF
    mesh = plsc.VectorSubcoreMesh(
        core_axis_name="c", subcore_axis_name="s",
        num_cores=_NC, num_subcores=_NS,
    )

    @functools.partial(
        pl.kernel,
        out_type=jax.ShapeDtypeStruct((b_tot, h), jnp.float32),
        mesh=mesh,
        scratch_types=[
            pltpu.VMEM((nch, k), jnp.int32),
            pltpu.VMEM((nbuf, k, h), jnp.float32),
        ] + [pltpu.SemaphoreType.DMA] * (2 * nbuf),
    )
    def body(idx_hbm, table_hbm, out_hbm, idx_v, buf, *sems):
        gsems = sems[:nbuf]
        wsems = sems[nbuf:]
        wid = lax.axis_index("s") * _NC + lax.axis_index("c")
        base = wid * (nch * k)
        pltpu.sync_copy(idx_hbm.at[wid], idx_v)

        def gather_start(j, p):
            pltpu.async_copy(table_hbm.at[idx_v.at[j]], buf.at[p], gsems[p])

        def gather_wait(j, p):
            pltpu.make_async_copy(
                table_hbm.at[idx_v.at[j]], buf.at[p], gsems[p]).wait()

        def out_slice(j):
            return out_hbm.at[pl.ds(base + j * k, k)]

        def write_start(j, p):
            pltpu.async_copy(buf.at[p], out_slice(j), wsems[p])

        def write_wait(j, p):
            pltpu.make_async_copy(buf.at[p], out_slice(j), wsems[p]).wait()

        gather_start(0, 0)
        gather_start(1, 1)

        def loop_body(jj, carry):
            j0 = jj * nbuf
            for p in range(nbuf):
                j = j0 + p
                gather_wait(j, p)
                write_start(j, p)

                @pl.when((j >= 2) & (j + 2 < nch))
                def _():
                    write_wait(j - 2, (p + nbuf - 2) % nbuf)

                @pl.when(j + 2 < nch)
                def _():
                    gather_start(j + 2, (p + 2) % nbuf)
            return carry

        lax.fori_loop(0, nch // nbuf, loop_body, 0)
        for t in range(4):
            j = nch - 4 + t
            write_wait(j, j % nbuf)

    return body


def kernel(input_ids, word_embeddings):
    b, s = input_ids.shape
    v, h = word_embeddings.shape
    b_tot = b * s
    b_per_w = b_tot // _NW
    nch = b_per_w // _K
    idx = jnp.transpose(input_ids.astype(jnp.int32)).reshape(_NW, nch, _K)
    table = word_embeddings.astype(jnp.float32)
    out = _build_sc_gather(b_tot, v, h, _K, nch)(idx, table)
    return out.reshape(s, b, h)


# speedup vs baseline: 2.4740x; 2.4740x over previous
import functools

import jax
import jax.numpy as jnp
from jax import lax
from jax.experimental import pallas as pl
from jax.experimental.pallas import tpu as pltpu
from jax.experimental.pallas import tpu_sc as plsc

_NC = 2
_NS = 16
_NW = _NC * _NS

_K = 4
_NBUF = 4


@functools.lru_cache(maxsize=None)
def _build_sc_gather(seq, bsz, v, h, k, nch):
    assert k == bsz
    nbuf = _NBUF
    mesh = plsc.VectorSubcoreMesh(
        core_axis_name="c", subcore_axis_name="s",
        num_cores=_NC, num_subcores=_NS,
    )

    @functools.partial(
        pl.kernel,
        out_type=jax.ShapeDtypeStruct((seq, bsz, h), jnp.float32),
        mesh=mesh,
        scratch_types=[
            pltpu.VMEM((nch, k), jnp.int32),
            pltpu.VMEM((nbuf, k, h), jnp.float32),
        ] + [pltpu.SemaphoreType.DMA] * (2 * nbuf),
    )
    def body(idx_hbm, table_hbm, out_hbm, idx_v, buf, *sems):
        gsems = sems[:nbuf]
        wsems = sems[nbuf:]
        wid = lax.axis_index("s") * _NC + lax.axis_index("c")
        sbase = wid * nch
        pltpu.sync_copy(idx_hbm.at[wid], idx_v)

        def gather_start(j, p):
            pltpu.async_copy(table_hbm.at[idx_v.at[j]], buf.at[p], gsems[p])

        def gather_wait(j, p):
            pltpu.make_async_copy(
                table_hbm.at[idx_v.at[j]], buf.at[p], gsems[p]).wait()

        def out_slice(j):
            return out_hbm.at[sbase + j]

        def write_start(j, p):
            pltpu.async_copy(buf.at[p], out_slice(j), wsems[p])

        def write_wait(j, p):
            pltpu.make_async_copy(buf.at[p], out_slice(j), wsems[p]).wait()

        gather_start(0, 0)
        gather_start(1, 1)

        def loop_body(jj, carry):
            j0 = jj * nbuf
            for p in range(nbuf):
                j = j0 + p
                gather_wait(j, p)
                write_start(j, p)

                @pl.when((j >= 2) & (j + 2 < nch))
                def _():
                    write_wait(j - 2, (p + nbuf - 2) % nbuf)

                @pl.when(j + 2 < nch)
                def _():
                    gather_start(j + 2, (p + 2) % nbuf)
            return carry

        lax.fori_loop(0, nch // nbuf, loop_body, 0)
        for t in range(4):
            j = nch - 4 + t
            write_wait(j, j % nbuf)

    return body


def kernel(input_ids, word_embeddings):
    b, s = input_ids.shape
    v, h = word_embeddings.shape
    k = b
    nch = s // _NW
    idx = jnp.transpose(input_ids.astype(jnp.int32)).reshape(_NW, nch, k)
    table = word_embeddings.astype(jnp.float32)
    return _build_sc_gather(s, b, v, h, k, nch)(idx, table)


# speedup vs baseline: 2.4755x; 1.0006x over previous
import functools

import jax
import jax.numpy as jnp
from jax import lax
from jax.experimental import pallas as pl
from jax.experimental.pallas import tpu as pltpu
from jax.experimental.pallas import tpu_sc as plsc

_NC = 2
_NS = 16
_NW = _NC * _NS

_K = 4
_NBUF = 6


@functools.lru_cache(maxsize=None)
def _build_sc_gather(seq, bsz, v, h, k, nch):
    assert k == bsz
    nbuf = _NBUF
    mesh = plsc.VectorSubcoreMesh(
        core_axis_name="c", subcore_axis_name="s",
        num_cores=_NC, num_subcores=_NS,
    )

    @functools.partial(
        pl.kernel,
        out_type=jax.ShapeDtypeStruct((seq, bsz, h), jnp.float32),
        mesh=mesh,
        scratch_types=[
            pltpu.VMEM((nch, k), jnp.int32),
            pltpu.VMEM((nbuf, k, h), jnp.float32),
        ] + [pltpu.SemaphoreType.DMA] * (2 * nbuf),
    )
    def body(idx_hbm, table_hbm, out_hbm, idx_v, buf, *sems):
        gsems = sems[:nbuf]
        wsems = sems[nbuf:]
        wid = lax.axis_index("s") * _NC + lax.axis_index("c")
        sbase = wid * nch
        pltpu.sync_copy(idx_hbm.at[wid], idx_v)

        def gather_start(j, p):
            pltpu.async_copy(table_hbm.at[idx_v.at[j]], buf.at[p], gsems[p])

        def gather_wait(j, p):
            pltpu.make_async_copy(
                table_hbm.at[idx_v.at[j]], buf.at[p], gsems[p]).wait()

        def out_slice(j):
            return out_hbm.at[sbase + j]

        def write_start(j, p):
            pltpu.async_copy(buf.at[p], out_slice(j), wsems[p])

        def write_wait(j, p):
            pltpu.make_async_copy(buf.at[p], out_slice(j), wsems[p]).wait()

        gather_start(0, 0)
        gather_start(1, 1)

        def visit(j, p):
            gather_wait(j, p)
            write_start(j, p)

            @pl.when((j >= nbuf - 2) & (j + 2 < nch))
            def _():
                write_wait(j + 2 - nbuf, (p + 2) % nbuf)

            @pl.when(j + 2 < nch)
            def _():
                gather_start(j + 2, (p + 2) % nbuf)

        def loop_body(jj, carry):
            j0 = jj * nbuf
            for p in range(nbuf):
                visit(j0 + p, p)
            return carry

        ngroups = nch // nbuf
        lax.fori_loop(0, ngroups, loop_body, 0)
        for j in range(ngroups * nbuf, nch):
            visit(j, j % nbuf)
        for j in range(nch - nbuf, nch):
            write_wait(j, j % nbuf)

    return body


def kernel(input_ids, word_embeddings):
    b, s = input_ids.shape
    v, h = word_embeddings.shape
    k = b
    nch = s // _NW
    idx = jnp.transpose(input_ids.astype(jnp.int32)).reshape(_NW, nch, k)
    table = word_embeddings.astype(jnp.float32)
    return _build_sc_gather(s, b, v, h, k, nch)(idx, table)
